# Initial kernel scaffold; baseline (speedup 1.0000x reference)
import jax
import jax.numpy as jnp
from jax.experimental import pallas as pl


def kernel(qs, types, id_table, que_table, que_W, que_b, ana_table, ana_W, ana_b, type_table):
    raise NotImplementedError("write your pallas kernel here")


# speedup vs baseline: 2.1051x; 2.1051x over previous
import functools

import jax
import jax.numpy as jnp
from jax import lax
from jax.experimental import pallas as pl
from jax.experimental.pallas import tpu as pltpu, tpu_sc as plsc

EMB = 64
PRETRAIN = 768


def _proj_body(que_ref, ana_ref, qW_ref, qb_ref, aW_ref, ab_ref, oq_ref, oa_ref):
    oq_ref[...] = (
        jnp.dot(que_ref[...], qW_ref[...], preferred_element_type=jnp.float32)
        + qb_ref[...]
    )
    oa_ref[...] = (
        jnp.dot(ana_ref[...], aW_ref[...], preferred_element_type=jnp.float32)
        + ab_ref[...]
    )


def _project(que_table, que_W, que_b, ana_table, ana_W, ana_b):
    rows = que_table.shape[0]
    rb = 2000
    assert rows % rb == 0
    return pl.pallas_call(
        _proj_body,
        grid=(rows // rb,),
        in_specs=[
            pl.BlockSpec((rb, PRETRAIN), lambda i: (i, 0)),
            pl.BlockSpec((rb, PRETRAIN), lambda i: (i, 0)),
            pl.BlockSpec((PRETRAIN, EMB), lambda i: (0, 0)),
            pl.BlockSpec((1, EMB), lambda i: (0, 0)),
            pl.BlockSpec((PRETRAIN, EMB), lambda i: (0, 0)),
            pl.BlockSpec((1, EMB), lambda i: (0, 0)),
        ],
        out_specs=[
            pl.BlockSpec((rb, EMB), lambda i: (i, 0)),
            pl.BlockSpec((rb, EMB), lambda i: (i, 0)),
        ],
        out_shape=[
            jax.ShapeDtypeStruct((rows, EMB), jnp.float32),
            jax.ShapeDtypeStruct((rows, EMB), jnp.float32),
        ],
    )(que_table, ana_table, que_W, que_b.reshape(1, EMB), ana_W, ana_b.reshape(1, EMB))


@functools.lru_cache(maxsize=None)
def _make_gather(ntok):
    info = plsc.get_sparse_core_info()
    nc, ns = info.num_cores, info.num_subcores
    nw = nc * ns
    assert ntok % nw == 0
    per_w = ntok // nw
    chunk = 128
    assert per_w % chunk == 0
    nch = per_w // chunk

    mesh = plsc.VectorSubcoreMesh(core_axis_name="c", subcore_axis_name="s")

    @functools.partial(
        pl.kernel,
        mesh=mesh,
        compiler_params=pltpu.CompilerParams(use_tc_tiling_on_sc=False),
        out_type=[jax.ShapeDtypeStruct((ntok, EMB), jnp.float32) for _ in range(4)],
        scratch_types=[
            pltpu.VMEM((chunk,), jnp.int32),
            pltpu.VMEM((chunk,), jnp.int32),
            pltpu.VMEM((chunk, EMB), jnp.float32),
            pltpu.VMEM((chunk, EMB), jnp.float32),
            pltpu.VMEM((chunk, EMB), jnp.float32),
            pltpu.VMEM((chunk, EMB), jnp.float32),
            pltpu.SemaphoreType.DMA,
        ],
    )
    def gather_k(qs_hbm, types_hbm, id_hbm, que_hbm, ana_hbm, type_hbm,
                 o_id, o_que, o_ana, o_type,
                 idx_v, tidx_v, r_id, r_que, r_ana, r_type, sem):
        wid = lax.axis_index("s") * nc + lax.axis_index("c")
        base = wid * per_w

        def body(c, carry):
            off = base + c * chunk
            pltpu.sync_copy(qs_hbm.at[pl.ds(off, chunk)], idx_v)
            pltpu.sync_copy(types_hbm.at[pl.ds(off, chunk)], tidx_v)
            cp1 = pltpu.async_copy(id_hbm.at[idx_v], r_id, sem)
            cp2 = pltpu.async_copy(que_hbm.at[idx_v], r_que, sem)
            cp3 = pltpu.async_copy(ana_hbm.at[idx_v], r_ana, sem)
            cp4 = pltpu.async_copy(type_hbm.at[tidx_v], r_type, sem)
            cp1.wait()
            cp2.wait()
            cp3.wait()
            cp4.wait()
            pltpu.sync_copy(r_id, o_id.at[pl.ds(off, chunk)])
            pltpu.sync_copy(r_que, o_que.at[pl.ds(off, chunk)])
            pltpu.sync_copy(r_ana, o_ana.at[pl.ds(off, chunk)])
            pltpu.sync_copy(r_type, o_type.at[pl.ds(off, chunk)])
            return carry

        lax.fori_loop(0, nch, body, 0)

    return gather_k


def kernel(qs, types, id_table, que_table, que_W, que_b, ana_table, ana_W, ana_b, type_table):
    b, l = qs.shape
    ntok = b * l
    proj_que, proj_ana = _project(que_table, que_W, que_b, ana_table, ana_W, ana_b)
    gather = _make_gather(ntok)
    o_id, o_que, o_ana, o_type = gather(
        qs.reshape(ntok), types.reshape(ntok),
        id_table, proj_que, proj_ana, type_table,
    )
    return (
        o_id.reshape(b, l, EMB),
        o_que.reshape(b, l, EMB),
        o_ana.reshape(b, l, EMB),
        o_type.reshape(b, l, EMB),
    )
